# baseline (speedup 1.0000x reference)
import jax
import jax.numpy as jnp
from jax.experimental import pallas as pl
from jax.experimental.pallas import tpu as pltpu

_CODEBOOK = 1024
_D = 64
_BETA = 0.25
_BB = 4


def _vq_body(z_ref, emb_ref, zq_ref, idx_ref, loss_ref, idxs_ref):
    z = jnp.concatenate([z_ref[i] for i in range(_BB)], axis=1)
    emb = emb_ref[...]
    s1 = z_ref.shape[2]
    s = _BB * s1

    m2ze = jax.lax.dot_general(
        emb * -2.0, z, (((1,), (0,)), ((), ())),
        preferred_element_type=jnp.float32)
    z2 = jnp.sum(z * z, axis=0)
    e2 = jnp.sum(emb * emb, axis=1)
    dist = (z2[None, :] + e2[:, None]) + m2ze

    mind = jnp.min(dist, axis=0)
    eqm = dist == mind[None, :]
    mask = eqm.astype(jnp.float32)

    ones = jnp.ones((1, _CODEBOOK), jnp.float32)
    io = jax.lax.broadcasted_iota(
        jnp.int32, (1, _CODEBOOK), 1).astype(jnp.float32)
    moments = jax.lax.dot_general(
        jnp.concatenate([ones, io, io * io], axis=0), mask,
        (((1,), (0,)), ((), ())),
        preferred_element_type=jnp.float32)
    cnt = moments[0:1]
    isum = moments[1:2]
    qsum = moments[2:3]
    low = (isum - jnp.sqrt(cnt * qsum - isum * isum)) / cnt
    idxs_ref[...] = low.astype(jnp.int32)

    @pl.when(jnp.max(cnt) >= 3.0)
    def _exact_ties():
        iota_i = jax.lax.broadcasted_iota(jnp.int32, (_CODEBOOK, s), 0)
        big = jnp.int32(_CODEBOOK)
        idxs_ref[...] = jnp.min(
            jnp.where(eqm, iota_i, big), axis=0)[None, :]

    idx = idxs_ref[0]
    iota_i = jax.lax.broadcasted_iota(jnp.int32, (_CODEBOOK, s), 0)
    onehot = (iota_i == idx[None, :]).astype(jnp.bfloat16)
    zq = jax.lax.dot_general(
        emb.astype(jnp.bfloat16), onehot, (((0,), (0,)), ((), ())),
        preferred_element_type=jnp.float32)
    for i in range(_BB):
        zq_ref[i] = zq[:, i * s1:(i + 1) * s1]
        idx_ref[i, 0, :] = idx[i * s1:(i + 1) * s1]

    part = jnp.sum(mind)

    @pl.when(pl.program_id(0) == 0)
    def _init():
        loss_ref[0, 0] = part

    @pl.when(pl.program_id(0) != 0)
    def _acc():
        loss_ref[0, 0] += part


def kernel(z_e, emb_weight):
    B, D, Gh, Gw = z_e.shape
    S = Gh * Gw
    z3 = z_e.reshape(B, D, S)

    zq3, idx3, loss_sum = pl.pallas_call(
        _vq_body,
        grid=(B // _BB,),
        in_specs=[
            pl.BlockSpec((_BB, D, S), lambda b: (b, 0, 0)),
            pl.BlockSpec((_CODEBOOK, D), lambda b: (0, 0)),
        ],
        out_specs=[
            pl.BlockSpec((_BB, D, S), lambda b: (b, 0, 0)),
            pl.BlockSpec((_BB, 1, S), lambda b: (b, 0, 0)),
            pl.BlockSpec((1, 1), lambda b: (0, 0), memory_space=pltpu.SMEM),
        ],
        out_shape=[
            jax.ShapeDtypeStruct((B, D, S), jnp.float32),
            jax.ShapeDtypeStruct((B, 1, S), jnp.int32),
            jax.ShapeDtypeStruct((1, 1), jnp.float32),
        ],
        scratch_shapes=[pltpu.VMEM((1, _BB * S), jnp.int32)],
    )(z3, emb_weight)

    z_q_st = zq3.reshape(B, D, Gh, Gw)
    idx = idx3.reshape(B, Gh, Gw)
    mean_sq = loss_sum[0, 0] / jnp.float32(B * S * D)
    vq_loss = mean_sq + _BETA * mean_sq
    return (z_q_st, idx, vq_loss)
